# baseline (speedup 1.0000x reference)
import functools

import jax
import jax.numpy as jnp
from jax import lax
from jax.experimental import pallas as pl
from jax.experimental.pallas import tpu as pltpu
from jax.experimental.pallas import tpu_sc as plsc

VOCAB = 1000000
EMBD = 64
BATCH = 4096
NEGS = 20480
TOT = BATCH + NEGS

NC, NS = 2, 16
NW = NC * NS
ROWS_B = BATCH // NW
ROWS_N = NEGS // NW
IDXW = 128
NCHUNK = ROWS_N // IDXW

CH = 512
GRID = TOT // CH
POS = BATCH // CH

@functools.cache
def _make_gather3():
    mesh = plsc.VectorSubcoreMesh(core_axis_name="c", subcore_axis_name="s")
    return functools.partial(
        pl.kernel,
        mesh=mesh,
        compiler_params=pltpu.CompilerParams(use_tc_tiling_on_sc=False),
        out_type=(
            jax.ShapeDtypeStruct((TOT, EMBD), jnp.float32),
            jax.ShapeDtypeStruct((BATCH, EMBD), jnp.float32),
        ),
        scratch_types=[
            pltpu.VMEM((ROWS_B,), jnp.int32),
            pltpu.VMEM((ROWS_B,), jnp.int32),
            pltpu.VMEM((ROWS_N,), jnp.int32),
            pltpu.VMEM((2 * ROWS_B + ROWS_N, EMBD), jnp.float32),
            pltpu.SemaphoreType.DMA,
        ],
    )(_gather3_body)


def _gather3_body(emb_hbm, ffw_hbm, inpt_h, trgs_h, rand_h, x_out, e_out,
                  idx_e, idx_c, idx_r, rows, sem):
    wid = lax.axis_index("s") * NC + lax.axis_index("c")
    pltpu.sync_copy(inpt_h.at[pl.ds(wid * ROWS_B, ROWS_B)], idx_e)
    pltpu.sync_copy(trgs_h.at[pl.ds(wid * ROWS_B, ROWS_B)], idx_c)
    pltpu.sync_copy(rand_h.at[pl.ds(wid * ROWS_N, ROWS_N)], idx_r)
    cps = [
        pltpu.async_copy(emb_hbm.at[idx_e], rows.at[pl.ds(0, ROWS_B)], sem),
        pltpu.async_copy(ffw_hbm.at[idx_c], rows.at[pl.ds(ROWS_B, ROWS_B)], sem),
    ]
    for j in range(NCHUNK):
        cps.append(pltpu.async_copy(
            ffw_hbm.at[idx_r.at[pl.ds(j * IDXW, IDXW)]],
            rows.at[pl.ds(2 * ROWS_B + j * IDXW, IDXW)], sem))
    for c in cps:
        c.wait()
    pltpu.sync_copy(rows.at[pl.ds(0, ROWS_B)],
                    e_out.at[pl.ds(wid * ROWS_B, ROWS_B)])
    pltpu.sync_copy(rows.at[pl.ds(ROWS_B, ROWS_B)],
                    x_out.at[pl.ds(wid * ROWS_B, ROWS_B)])
    pltpu.sync_copy(rows.at[pl.ds(2 * ROWS_B, ROWS_N)],
                    x_out.at[pl.ds(BATCH + wid * ROWS_N, ROWS_N)])


def _loss_body(x_ref, e_ref, o_ref):
    i = pl.program_id(0)
    t = lax.dot_general(x_ref[...], e_ref[...], (((1,), (1,)), ((), ())),
                        preferred_element_type=jnp.float32)
    s = jnp.clip(jax.nn.sigmoid(t), 1e-07, 1.0 - 1e-07)
    is_pos = i < POS
    sel = jnp.where(is_pos, s, 1.0 - s)
    part = jnp.sum(-jnp.log(sel))
    w = jnp.where(is_pos, 1.0 / (BATCH * BATCH), 1.0 / (NEGS * BATCH))

    @pl.when(i == 0)
    def _init():
        o_ref[0, 0] = 0.0

    o_ref[0, 0] += part * w


_loss_call = pl.pallas_call(
    _loss_body,
    grid=(GRID,),
    in_specs=[
        pl.BlockSpec((CH, EMBD), lambda i: (i, 0)),
        pl.BlockSpec((BATCH, EMBD), lambda i: (0, 0)),
    ],
    out_specs=pl.BlockSpec(memory_space=pltpu.SMEM),
    out_shape=jax.ShapeDtypeStruct((1, 1), jnp.float32),
)


def kernel(inpt, trgs, rand, emb_table, ffw_weight):
    x_all, e_all = _make_gather3()(
        emb_table, ffw_weight,
        inpt.astype(jnp.int32), trgs.astype(jnp.int32), rand.astype(jnp.int32))
    loss = _loss_call(x_all, e_all)
    return loss[0, 0]
